# baseline (speedup 1.0000x reference)
import functools

import jax
import jax.numpy as jnp
from jax import lax
from jax.experimental import pallas as pl
from jax.experimental.pallas import tpu as pltpu
from jax.experimental.pallas import tpu_sc as plsc

_NC = 2
_NS = 16
_NW = _NC * _NS


def _gather_sc(table, idx_flat, B, D, CH, NBUF=2):
    b_per_w = B // _NW
    n_chunks = b_per_w // CH
    assert n_chunks % NBUF == 0
    mesh = plsc.VectorSubcoreMesh(core_axis_name="c", subcore_axis_name="s")

    @functools.partial(
        pl.kernel,
        mesh=mesh,
        out_type=jax.ShapeDtypeStruct((B, D), jnp.float32),
        scratch_types=[
            pltpu.VMEM((NBUF, CH), jnp.int32),
            pltpu.VMEM((NBUF, CH, D), jnp.float32),
            [pltpu.SemaphoreType.DMA] * NBUF,
            [pltpu.SemaphoreType.DMA] * NBUF,
        ],
        compiler_params=pltpu.CompilerParams(use_tc_tiling_on_sc=False),
    )
    def k(table_hbm, idx_hbm, out_hbm, idx_v, rows_v, gsem, ssem):
        wid = lax.axis_index("s") * _NC + lax.axis_index("c")
        base = wid * b_per_w

        def body(g, carry):
            for b in range(NBUF):
                off = base + (g * NBUF + b) * CH

                @pl.when(g > 0)
                def _():
                    pltpu.make_async_copy(
                        rows_v.at[b], out_hbm.at[pl.ds(base, CH)], ssem[b]
                    ).wait()

                pltpu.sync_copy(idx_hbm.at[pl.ds(off, CH)], idx_v.at[b])
                pltpu.async_copy(
                    table_hbm.at[idx_v.at[b]], rows_v.at[b], gsem[b]
                ).wait()
                pltpu.async_copy(rows_v.at[b], out_hbm.at[pl.ds(off, CH)], ssem[b])
            return carry

        lax.fori_loop(0, n_chunks // NBUF, body, 0)
        for b in range(NBUF):
            pltpu.make_async_copy(
                rows_v.at[b], out_hbm.at[pl.ds(base, CH)], ssem[b]
            ).wait()

    return k(table, idx_flat)


def kernel(table, indices):
    B0, B1 = indices.shape
    V, D = table.shape
    idx_flat = indices.reshape(-1).astype(jnp.int32)
    B = idx_flat.shape[0]
    out = _gather_sc(table, idx_flat, B, D, CH=1600)
    return out.reshape(B0, B1, D)


# speedup vs baseline: 1.4120x; 1.4120x over previous
import functools

import jax
import jax.numpy as jnp
from jax import lax
from jax.experimental import pallas as pl
from jax.experimental.pallas import tpu as pltpu
from jax.experimental.pallas import tpu_sc as plsc

_NC = 2
_NS = 16
_NW = _NC * _NS


def _gather_sc(table, idx_t, V, D, J, B):
    W = B // _NW
    NTB = W // 128
    NF = D // 8
    mesh = plsc.VectorSubcoreMesh(core_axis_name="c", subcore_axis_name="s")

    @functools.partial(
        pl.kernel,
        mesh=mesh,
        out_type=jax.ShapeDtypeStruct((J, NF, B // 128, 8, 128), jnp.float32),
        scratch_types=[
            pltpu.VMEM((2, W), jnp.int32),
            pltpu.VMEM((2, W, D), jnp.float32),
            pltpu.VMEM((2, NF, 8, W), jnp.float32),
            [pltpu.SemaphoreType.DMA] * 2,
            [pltpu.SemaphoreType.DMA] * 2,
        ],
        compiler_params=pltpu.CompilerParams(
            use_tc_tiling_on_sc=False, needs_layout_passes=False
        ),
    )
    def k(table_hbm, idx_hbm, out_hbm, idx_v, rows_v, trans_v, gsem, wsem):
        wid = lax.axis_index("s") * _NC + lax.axis_index("c")
        bstart = wid * W
        lane = lax.iota(jnp.int32, 16)

        pltpu.sync_copy(idx_hbm.at[0, pl.ds(bstart, W)], idx_v.at[0])
        pltpu.async_copy(table_hbm.at[idx_v.at[0]], rows_v.at[0], gsem[0])

        def slot(j, p):
            pltpu.make_async_copy(
                table_hbm.at[idx_v.at[p]], rows_v.at[p], gsem[p]
            ).wait()

            @pl.when(j + 1 < J)
            def _():
                pltpu.sync_copy(
                    idx_hbm.at[j + 1, pl.ds(bstart, W)], idx_v.at[1 - p]
                )
                pltpu.async_copy(
                    table_hbm.at[idx_v.at[1 - p]], rows_v.at[1 - p],
                    gsem[1 - p],
                )

            @pl.when(j >= 2)
            def _():
                for tf in range(NF):
                    for t in range(NTB):
                        pltpu.make_async_copy(
                            trans_v.at[p, tf, :, pl.ds(t * 128, 128)],
                            out_hbm.at[0, tf, t, :, :],
                            wsem[p],
                        ).wait()

            def tstep(s, carry):
                ridx = s * 16 + lane
                for tf in range(NF):
                    for f in range(8):
                        cidx = jnp.full((16,), tf * 8 + f, jnp.int32)
                        vals = plsc.load_gather(rows_v.at[p], [ridx, cidx])
                        trans_v[p, tf, f, pl.ds(s * 16, 16)] = vals
                return carry

            lax.fori_loop(0, W // 16, tstep, 0)

            for tf in range(NF):
                for t in range(NTB):
                    pltpu.async_copy(
                        trans_v.at[p, tf, :, pl.ds(t * 128, 128)],
                        out_hbm.at[j, tf, wid * NTB + t, :, :],
                        wsem[p],
                    )

        def body(jj, carry):
            for p in range(2):
                slot(jj * 2 + p, p)
            return carry

        lax.fori_loop(0, J // 2, body, 0)

        for p in range(2):
            for tf in range(NF):
                for t in range(NTB):
                    pltpu.make_async_copy(
                        trans_v.at[p, tf, :, pl.ds(t * 128, 128)],
                        out_hbm.at[0, tf, t, :, :],
                        wsem[p],
                    ).wait()

    return k(table, idx_t)


def kernel(table, indices):
    B, J = indices.shape
    V, D = table.shape
    idx_t = jnp.transpose(indices).astype(jnp.int32)
    out5 = _gather_sc(table, idx_t, V, D, J, B)
    return out5.transpose(2, 4, 0, 1, 3).reshape(B, J, D)


# speedup vs baseline: 1.6905x; 1.1972x over previous
import functools

import jax
import jax.numpy as jnp
from jax import lax
from jax.experimental import pallas as pl
from jax.experimental.pallas import tpu as pltpu
from jax.experimental.pallas import tpu_sc as plsc

_NC = 2
_NS = 16
_NW = _NC * _NS


def _gather_sc(table, idx_t, V, D, J, B):
    W = B // _NW
    NTB = W // 128
    NF = D // 8
    mesh = plsc.VectorSubcoreMesh(core_axis_name="c", subcore_axis_name="s")

    @functools.partial(
        pl.kernel,
        mesh=mesh,
        out_type=jax.ShapeDtypeStruct((J, NF, B // 128, 8, 128), jnp.float32),
        scratch_types=[
            pltpu.VMEM((2, W), jnp.int32),
            pltpu.VMEM((2, W, D), jnp.float32),
            pltpu.VMEM((2, NF, 8, W), jnp.float32),
            [pltpu.SemaphoreType.DMA] * 2,
            [pltpu.SemaphoreType.DMA] * 2,
        ],
        compiler_params=pltpu.CompilerParams(
            use_tc_tiling_on_sc=False, needs_layout_passes=False
        ),
    )
    def k(table_hbm, idx_hbm, out_hbm, idx_v, rows_v, trans_v, gsem, wsem):
        wid = lax.axis_index("s") * _NC + lax.axis_index("c")
        bstart = wid * W
        lane = lax.iota(jnp.int32, 16)

        pltpu.sync_copy(idx_hbm.at[0, pl.ds(bstart, W)], idx_v.at[0])
        pltpu.async_copy(table_hbm.at[idx_v.at[0]], rows_v.at[0], gsem[0])

        def slot(j, p):
            pltpu.make_async_copy(
                table_hbm.at[idx_v.at[p]], rows_v.at[p], gsem[p]
            ).wait()

            @pl.when(j + 1 < J)
            def _():
                pltpu.sync_copy(
                    idx_hbm.at[j + 1, pl.ds(bstart, W)], idx_v.at[1 - p]
                )
                pltpu.async_copy(
                    table_hbm.at[idx_v.at[1 - p]], rows_v.at[1 - p],
                    gsem[1 - p],
                )

            @pl.when(j >= 2)
            def _():
                for tf in range(NF):
                    for t in range(NTB):
                        pltpu.make_async_copy(
                            trans_v.at[p, tf, :, pl.ds(t * 128, 128)],
                            out_hbm.at[0, tf, t, :, :],
                            wsem[p],
                        ).wait()

            @plsc.parallel_loop(0, W // 16, unroll=4)
            def tstep(s):
                ridx = s * 16 + lane
                for tf in range(NF):
                    for f in range(8):
                        cidx = jnp.full((16,), tf * 8 + f, jnp.int32)
                        vals = plsc.load_gather(rows_v.at[p], [ridx, cidx])
                        trans_v[p, tf, f, pl.ds(s * 16, 16)] = vals

            for tf in range(NF):
                for t in range(NTB):
                    pltpu.async_copy(
                        trans_v.at[p, tf, :, pl.ds(t * 128, 128)],
                        out_hbm.at[j, tf, wid * NTB + t, :, :],
                        wsem[p],
                    )

        def body(jj, carry):
            for p in range(2):
                slot(jj * 2 + p, p)
            return carry

        lax.fori_loop(0, J // 2, body, 0)

        for p in range(2):
            for tf in range(NF):
                for t in range(NTB):
                    pltpu.make_async_copy(
                        trans_v.at[p, tf, :, pl.ds(t * 128, 128)],
                        out_hbm.at[0, tf, t, :, :],
                        wsem[p],
                    ).wait()

    return k(table, idx_t)


def kernel(table, indices):
    B, J = indices.shape
    V, D = table.shape
    idx_t = jnp.transpose(indices).astype(jnp.int32)
    out5 = _gather_sc(table, idx_t, V, D, J, B)
    return out5.transpose(2, 4, 0, 1, 3).reshape(B, J, D)


# speedup vs baseline: 2.6960x; 1.5948x over previous
import functools

import jax
import jax.numpy as jnp
from jax import lax
from jax.experimental import pallas as pl
from jax.experimental.pallas import tpu as pltpu
from jax.experimental.pallas import tpu_sc as plsc

_NC = 2
_NS = 16
_NW = _NC * _NS


def _gather_sc(table, idx_t, V, D, J, B):
    W = B // _NW
    NTB = W // 128
    NF = D // 8
    mesh = plsc.VectorSubcoreMesh(core_axis_name="c", subcore_axis_name="s")

    @functools.partial(
        pl.kernel,
        mesh=mesh,
        out_type=jax.ShapeDtypeStruct((J, NF, B // 128, 8, 128), jnp.float32),
        scratch_types=[
            pltpu.VMEM((2, W), jnp.int32),
            pltpu.VMEM((2, W, D), jnp.float32),
            pltpu.VMEM((2, D, W + 1), jnp.float32),
            [pltpu.SemaphoreType.DMA] * 2,
            [pltpu.SemaphoreType.DMA] * 2,
        ],
        compiler_params=pltpu.CompilerParams(
            use_tc_tiling_on_sc=False, needs_layout_passes=False
        ),
    )
    def k(table_hbm, idx_hbm, out_hbm, idx_v, rows_v, trans_v, gsem, wsem):
        wid = lax.axis_index("s") * _NC + lax.axis_index("c")
        bstart = wid * W
        lane = lax.iota(jnp.int32, 16)

        pltpu.sync_copy(idx_hbm.at[0, pl.ds(bstart, W)], idx_v.at[0])
        pltpu.async_copy(table_hbm.at[idx_v.at[0]], rows_v.at[0], gsem[0])

        def slot(j, p):
            pltpu.make_async_copy(
                table_hbm.at[idx_v.at[p]], rows_v.at[p], gsem[p]
            ).wait()

            @pl.when(j + 1 < J)
            def _():
                pltpu.sync_copy(
                    idx_hbm.at[j + 1, pl.ds(bstart, W)], idx_v.at[1 - p]
                )
                pltpu.async_copy(
                    table_hbm.at[idx_v.at[1 - p]], rows_v.at[1 - p],
                    gsem[1 - p],
                )

            @pl.when(j >= 2)
            def _():
                for tf in range(NF):
                    for t in range(NTB):
                        pltpu.make_async_copy(
                            trans_v.at[p, pl.ds(tf * 8, 8), pl.ds(t * 128, 128)],
                            out_hbm.at[0, tf, t, :, :],
                            wsem[p],
                        ).wait()

            @plsc.parallel_loop(0, W, unroll=4)
            def tstep(b):
                for g in range(D // 16):
                    fidx = g * 16 + lane
                    vals = rows_v[p, b, pl.ds(g * 16, 16)]
                    bidx = jnp.full((16,), 0, jnp.int32) + b
                    plsc.store_scatter(trans_v.at[p], [fidx, bidx], vals)

            for tf in range(NF):
                for t in range(NTB):
                    pltpu.async_copy(
                        trans_v.at[p, pl.ds(tf * 8, 8), pl.ds(t * 128, 128)],
                        out_hbm.at[j, tf, wid * NTB + t, :, :],
                        wsem[p],
                    )

        def body(jj, carry):
            for p in range(2):
                slot(jj * 2 + p, p)
            return carry

        lax.fori_loop(0, J // 2, body, 0)

        for p in range(2):
            for tf in range(NF):
                for t in range(NTB):
                    pltpu.make_async_copy(
                        trans_v.at[p, pl.ds(tf * 8, 8), pl.ds(t * 128, 128)],
                        out_hbm.at[0, tf, t, :, :],
                        wsem[p],
                    ).wait()

    return k(table, idx_t)


def kernel(table, indices):
    B, J = indices.shape
    V, D = table.shape
    idx_t = jnp.transpose(indices).astype(jnp.int32)
    out5 = _gather_sc(table, idx_t, V, D, J, B)
    return out5.transpose(2, 4, 0, 1, 3).reshape(B, J, D)


# speedup vs baseline: 2.7000x; 1.0015x over previous
import functools

import jax
import jax.numpy as jnp
from jax import lax
from jax.experimental import pallas as pl
from jax.experimental.pallas import tpu as pltpu
from jax.experimental.pallas import tpu_sc as plsc

_NC = 2
_NS = 16
_NW = _NC * _NS


def _gather_sc(table, idx_t, V, D, J, B):
    W = B // _NW
    NTB = W // 128
    NF = D // 8
    mesh = plsc.VectorSubcoreMesh(core_axis_name="c", subcore_axis_name="s")

    @functools.partial(
        pl.kernel,
        mesh=mesh,
        out_type=jax.ShapeDtypeStruct((J, NF, B // 128, 8, 128), jnp.float32),
        scratch_types=[
            pltpu.VMEM((2, W), jnp.int32),
            pltpu.VMEM((2, W, D), jnp.float32),
            pltpu.VMEM((2, D, W + 1), jnp.float32),
            [pltpu.SemaphoreType.DMA] * 2,
            [pltpu.SemaphoreType.DMA] * 2,
        ],
        compiler_params=pltpu.CompilerParams(
            use_tc_tiling_on_sc=False, needs_layout_passes=False
        ),
    )
    def k(table_hbm, idx_hbm, out_hbm, idx_v, rows_v, trans_v, gsem, wsem):
        wid = lax.axis_index("s") * _NC + lax.axis_index("c")
        bstart = wid * W
        lane = lax.iota(jnp.int32, 16)

        pltpu.sync_copy(idx_hbm.at[0, pl.ds(bstart, W)], idx_v.at[0])
        pltpu.async_copy(table_hbm.at[idx_v.at[0]], rows_v.at[0], gsem[0])

        def slot(j, p):
            pltpu.make_async_copy(
                table_hbm.at[idx_v.at[p]], rows_v.at[p], gsem[p]
            ).wait()

            @pl.when(j + 1 < J)
            def _():
                pltpu.sync_copy(
                    idx_hbm.at[j + 1, pl.ds(bstart, W)], idx_v.at[1 - p]
                )
                pltpu.async_copy(
                    table_hbm.at[idx_v.at[1 - p]], rows_v.at[1 - p],
                    gsem[1 - p],
                )

            @pl.when(j >= 2)
            def _():
                for tf in range(NF):
                    for t in range(NTB):
                        pltpu.make_async_copy(
                            trans_v.at[p, pl.ds(tf * 8, 8), pl.ds(t * 128, 128)],
                            out_hbm.at[0, tf, t, :, :],
                            wsem[p],
                        ).wait()

            @plsc.parallel_loop(0, W, unroll=8)
            def tstep(b):
                for g in range(D // 16):
                    fidx = g * 16 + lane
                    vals = rows_v[p, b, pl.ds(g * 16, 16)]
                    bidx = jnp.full((16,), 0, jnp.int32) + b
                    plsc.store_scatter(trans_v.at[p], [fidx, bidx], vals)

            for tf in range(NF):
                for t in range(NTB):
                    pltpu.async_copy(
                        trans_v.at[p, pl.ds(tf * 8, 8), pl.ds(t * 128, 128)],
                        out_hbm.at[j, tf, wid * NTB + t, :, :],
                        wsem[p],
                    )

        def body(jj, carry):
            for p in range(2):
                slot(jj * 2 + p, p)
            return carry

        lax.fori_loop(0, J // 2, body, 0)

        for p in range(2):
            for tf in range(NF):
                for t in range(NTB):
                    pltpu.make_async_copy(
                        trans_v.at[p, pl.ds(tf * 8, 8), pl.ds(t * 128, 128)],
                        out_hbm.at[0, tf, t, :, :],
                        wsem[p],
                    ).wait()

    return k(table, idx_t)


def kernel(table, indices):
    B, J = indices.shape
    V, D = table.shape
    idx_t = jnp.transpose(indices).astype(jnp.int32)
    out5 = _gather_sc(table, idx_t, V, D, J, B)
    return out5.transpose(2, 4, 0, 1, 3).reshape(B, J, D)
